# baseline (speedup 1.0000x reference)
import dataclasses

import jax
import jax.numpy as jnp
from jax import lax
from jax.experimental import pallas as pl
from jax.experimental.pallas import tpu as pltpu
from jax.experimental.pallas import tpu_sc as plsc

DIM = 64
BATCH = 16384
REG_COEF = 1e-05
NROWS = 100000
NC = 2
NS = 16
LANES = 16
NW = NC * NS
BPW = BATCH // NW
CHUNK = 128
NCHUNK = BPW // CHUNK
GPC = CHUNK // LANES

TBLK = 512
HALF = TBLK // 2
NTBLK = (NROWS + TBLK - 1) // TBLK
NOUT = NTBLK * HALF
WPR = DIM // 2


def _pack_blk(t):
    ii = lax.broadcasted_iota(jnp.int32, (DIM, DIM), 0)
    jj = lax.broadcasted_iota(jnp.int32, (DIM, DIM), 1)
    eye = (ii == jj).astype(jnp.bfloat16)
    a = lax.dot_general(t.astype(jnp.bfloat16), eye, (((0,), (0,)), ((), ())),
                        preferred_element_type=jnp.float32)
    a = a.astype(jnp.bfloat16)
    return jnp.concatenate([a[0:HALF, :], a[HALF:TBLK, :]], axis=1)


def _tr_body(ttu_ref, tti_ref, outu_ref, outi_ref):
    outu_ref[...] = _pack_blk(ttu_ref[...])
    outi_ref[...] = _pack_blk(tti_ref[...])


def _to_packed(user_table, item_table):
    return pl.pallas_call(
        _tr_body,
        grid=(NTBLK,),
        in_specs=[
            pl.BlockSpec((DIM, TBLK), lambda i: (0, i)),
            pl.BlockSpec((DIM, TBLK), lambda i: (0, i)),
        ],
        out_specs=[
            pl.BlockSpec((HALF, 128), lambda i: (i, 0)),
            pl.BlockSpec((HALF, 128), lambda i: (i, 0)),
        ],
        out_shape=[
            jax.ShapeDtypeStruct((NOUT, 128), jnp.bfloat16),
            jax.ShapeDtypeStruct((NOUT, 128), jnp.bfloat16),
        ],
    )(user_table.T, item_table.T)


def _sc_body(gidx_u, gidx_p, gidx_n, colb_u, colb_p, colb_n,
             utab, itab, diff_hbm, sq_hbm,
             iu_v, ip_v, in_v, cu_v, cp_v, cn_v,
             ru0, ru1, rp0, rp1, rn0, rn1,
             scores_v, sq_v, sem0, sem1):
    wid = lax.axis_index("s") * NC + lax.axis_index("c")

    pltpu.sync_copy(gidx_u.at[wid], iu_v)
    pltpu.sync_copy(gidx_p.at[wid], ip_v)
    pltpu.sync_copy(gidx_n.at[wid], in_v)
    pltpu.sync_copy(colb_u.at[wid], cu_v)
    pltpu.sync_copy(colb_p.at[wid], cp_v)
    pltpu.sync_copy(colb_n.at[wid], cn_v)

    rbufs = [(ru0, rp0, rn0), (ru1, rp1, rn1)]
    sems = [sem0, sem1]

    def fire(c):
        ru, rp, rn = rbufs[c % 2]
        sem = sems[c % 2]
        return [
            pltpu.async_copy(utab.at[iu_v.at[c]], ru, sem),
            pltpu.async_copy(itab.at[ip_v.at[c]], rp, sem),
            pltpu.async_copy(itab.at[in_v.at[c]], rn, sem),
        ]

    def pair(x):
        return plsc.unpack(plsc.bitcast(x, jnp.bfloat16),
                           format=plsc.PackFormat.INTERLEAVED)

    sq_v[...] = jnp.zeros((LANES,), jnp.float32)
    iota = lax.iota(jnp.int32, LANES)

    pending = fire(0)
    for c in range(NCHUNK):
        nxt = fire(c + 1) if c + 1 < NCHUNK else []
        for cpd in pending:
            cpd.wait()
        pending = nxt
        ru, rp, rn = rbufs[c % 2]

        @pl.loop(0, GPC)
        def _group(g):
            row = g * LANES + iota
            cu = cu_v[c, pl.ds(g * LANES, LANES)]
            cp_ = cp_v[c, pl.ds(g * LANES, LANES)]
            cn = cn_v[c, pl.ds(g * LANES, LANES)]
            pos = jnp.zeros((LANES,), jnp.float32)
            neg = jnp.zeros((LANES,), jnp.float32)
            sq = jnp.zeros((LANES,), jnp.float32)
            for t in range(WPR):
                ua, ub = pair(plsc.load_gather(ru, [row, cu + t]))
                pa, pb = pair(plsc.load_gather(rp, [row, cp_ + t]))
                na, nb = pair(plsc.load_gather(rn, [row, cn + t]))
                pos = pos + ua * pa + ub * pb
                neg = neg + ua * na + ub * nb
                sq = sq + (ua * ua + ub * ub + pa * pa + pb * pb
                           + na * na + nb * nb)
            scores_v[pl.ds(c * CHUNK + g * LANES, LANES)] = pos - neg
            sq_v[...] += sq

    pltpu.sync_copy(scores_v, diff_hbm.at[pl.ds(wid * BPW, BPW)])
    pltpu.sync_copy(sq_v, sq_hbm.at[wid])


def _loss_body(diff_ref, sq_ref, out_ref):
    d = diff_ref[...]
    sp = jnp.maximum(-d, 0.0) + jnp.log1p(jnp.exp(-jnp.abs(d)))
    out_ref[0] = jnp.sum(sp) * (1.0 / BATCH)
    out_ref[1] = (0.5 * REG_COEF) * jnp.sum(sq_ref[...])


@jax.jit
def kernel(userids, itemids_pos, itemids_neg, user_table, item_table):
    uid = userids.astype(jnp.int32)
    pid = itemids_pos.astype(jnp.int32)
    nid = itemids_neg.astype(jnp.int32)
    shp = (NW, NCHUNK, CHUNK)
    gidx = [((x // TBLK) * HALF + (x % HALF)).reshape(shp)
            for x in (uid, pid, nid)]
    colb = [(((x // HALF) & 1) * WPR).reshape(shp) for x in (uid, pid, nid)]

    utab_bf, itab_bf = _to_packed(user_table, item_table)
    utab = lax.bitcast_convert_type(
        utab_bf.reshape(NOUT, WPR * 2, 2), jnp.int32)
    itab = lax.bitcast_convert_type(
        itab_bf.reshape(NOUT, WPR * 2, 2), jnp.int32)

    mesh = plsc.VectorSubcoreMesh(
        core_axis_name="c", subcore_axis_name="s",
        num_cores=NC, num_subcores=NS)

    cp = pltpu.CompilerParams()
    if "needs_layout_passes" in pltpu.CompilerParams.__dataclass_fields__:
        cp = dataclasses.replace(cp, needs_layout_passes=False)
    if "use_tc_tiling_on_sc" in pltpu.CompilerParams.__dataclass_fields__:
        cp = dataclasses.replace(cp, use_tc_tiling_on_sc=False)

    idx_t = pltpu.VMEM((NCHUNK, CHUNK), jnp.int32)
    row_t = pltpu.VMEM((CHUNK, WPR * 2), jnp.int32)
    sc = pl.kernel(
        _sc_body,
        compiler_params=cp,
        out_type=[
            jax.ShapeDtypeStruct((BATCH,), jnp.float32),
            jax.ShapeDtypeStruct((NW, LANES), jnp.float32),
        ],
        mesh=mesh,
        scratch_types=[
            idx_t, idx_t, idx_t, idx_t, idx_t, idx_t,
            row_t, row_t, row_t, row_t, row_t, row_t,
            pltpu.VMEM((BPW,), jnp.float32),
            pltpu.VMEM((LANES,), jnp.float32),
            pltpu.SemaphoreType.DMA,
            pltpu.SemaphoreType.DMA,
        ],
    )
    diff, sq = sc(*gidx, *colb, utab, itab)

    out = pl.pallas_call(
        _loss_body,
        out_shape=jax.ShapeDtypeStruct((2,), jnp.float32),
        out_specs=pl.BlockSpec(memory_space=pltpu.SMEM),
    )(diff.reshape(BATCH // 128, 128), sq)
    return out[0], out[1]


# speedup vs baseline: 2.2380x; 2.2380x over previous
import dataclasses

import jax
import jax.numpy as jnp
from jax import lax
from jax.experimental import pallas as pl
from jax.experimental.pallas import tpu as pltpu
from jax.experimental.pallas import tpu_sc as plsc

DIM = 64
BATCH = 16384
REG_COEF = 1e-05
NROWS = 100000
NC = 2
NS = 16
LANES = 16
NW = NC * NS
BPW = BATCH // NW
CHUNK = 128
NCHUNK = BPW // CHUNK


def _sc_body(idx_u_h, idx_p_h, idx_n_h, utab, itab, diff_hbm, sq_hbm,
             iu_v, ip_v, in_v,
             ru0, ru1, rp0, rp1, rn0, rn1,
             scores_v, sq_v, sem0, sem1):
    wid = lax.axis_index("s") * NC + lax.axis_index("c")

    pltpu.sync_copy(idx_u_h.at[wid], iu_v)
    pltpu.sync_copy(idx_p_h.at[wid], ip_v)
    pltpu.sync_copy(idx_n_h.at[wid], in_v)

    rbufs = [(ru0, rp0, rn0), (ru1, rp1, rn1)]
    sems = [sem0, sem1]

    def fire(c):
        ru, rp, rn = rbufs[c % 2]
        sem = sems[c % 2]
        return [
            pltpu.async_copy(utab.at[iu_v.at[c]], ru, sem),
            pltpu.async_copy(itab.at[ip_v.at[c]], rp, sem),
            pltpu.async_copy(itab.at[in_v.at[c]], rn, sem),
        ]

    def halves(ref, r):
        a = ref[r, pl.ds(0, 32)]
        b = ref[r, pl.ds(32, 32)]
        a0, a1 = plsc.unpack(a, format=plsc.PackFormat.INTERLEAVED)
        b0, b1 = plsc.unpack(b, format=plsc.PackFormat.INTERLEAVED)
        return a0, a1, b0, b1

    sq_v[...] = jnp.zeros((LANES,), jnp.float32)

    pending = fire(0)
    for c in range(NCHUNK):
        nxt = fire(c + 1) if c + 1 < NCHUNK else []
        for cpd in pending:
            cpd.wait()
        pending = nxt
        ru, rp, rn = rbufs[c % 2]

        @pl.loop(0, CHUNK)
        def _row(r):
            u0, u1, u2, u3 = halves(ru, r)
            p0, p1, p2, p3 = halves(rp, r)
            n0, n1, n2, n3 = halves(rn, r)
            s = u0 * (p0 - n0) + u1 * (p1 - n1)
            s = s + u2 * (p2 - n2) + u3 * (p3 - n3)
            sq = (u0 * u0 + u1 * u1 + u2 * u2 + u3 * u3)
            sq = sq + (p0 * p0 + p1 * p1 + p2 * p2 + p3 * p3)
            sq = sq + (n0 * n0 + n1 * n1 + n2 * n2 + n3 * n3)
            scores_v[c * CHUNK + r] = s
            sq_v[...] += sq

    pltpu.sync_copy(scores_v, diff_hbm.at[pl.ds(wid * BPW, BPW)])
    pltpu.sync_copy(sq_v, sq_hbm.at[wid])


def _loss_body(diff_ref, sq_ref, out_ref):
    d = jnp.sum(diff_ref[...], axis=1)
    sp = jnp.maximum(-d, 0.0) + jnp.log1p(jnp.exp(-jnp.abs(d)))
    out_ref[0] = jnp.sum(sp) * (1.0 / BATCH)
    out_ref[1] = (0.5 * REG_COEF) * jnp.sum(sq_ref[...])


@jax.jit
def kernel(userids, itemids_pos, itemids_neg, user_table, item_table):
    shp = (NW, NCHUNK, CHUNK)
    gidx = [x.astype(jnp.int32).reshape(shp)
            for x in (userids, itemids_pos, itemids_neg)]

    utab = jnp.pad(user_table.astype(jnp.bfloat16), ((0, 0), (0, DIM)))
    itab = jnp.pad(item_table.astype(jnp.bfloat16), ((0, 0), (0, DIM)))

    mesh = plsc.VectorSubcoreMesh(
        core_axis_name="c", subcore_axis_name="s",
        num_cores=NC, num_subcores=NS)

    cp = pltpu.CompilerParams()
    if "needs_layout_passes" in pltpu.CompilerParams.__dataclass_fields__:
        cp = dataclasses.replace(cp, needs_layout_passes=False)
    if "use_tc_tiling_on_sc" in pltpu.CompilerParams.__dataclass_fields__:
        cp = dataclasses.replace(cp, use_tc_tiling_on_sc=False)

    idx_t = pltpu.VMEM((NCHUNK, CHUNK), jnp.int32)
    row_t = pltpu.VMEM((CHUNK, 2 * DIM), jnp.bfloat16)
    sc = pl.kernel(
        _sc_body,
        compiler_params=cp,
        out_type=[
            jax.ShapeDtypeStruct((BATCH, LANES), jnp.float32),
            jax.ShapeDtypeStruct((NW, LANES), jnp.float32),
        ],
        mesh=mesh,
        scratch_types=[
            idx_t, idx_t, idx_t,
            row_t, row_t, row_t, row_t, row_t, row_t,
            pltpu.VMEM((BPW, LANES), jnp.float32),
            pltpu.VMEM((LANES,), jnp.float32),
            pltpu.SemaphoreType.DMA,
            pltpu.SemaphoreType.DMA,
        ],
    )
    diff, sq = sc(*gidx, utab, itab)

    out = pl.pallas_call(
        _loss_body,
        out_shape=jax.ShapeDtypeStruct((2,), jnp.float32),
        out_specs=pl.BlockSpec(memory_space=pltpu.SMEM),
    )(diff, sq)
    return out[0], out[1]


# speedup vs baseline: 3.6693x; 1.6395x over previous
import dataclasses

import jax
import jax.numpy as jnp
from jax import lax
from jax.experimental import pallas as pl
from jax.experimental.pallas import tpu as pltpu
from jax.experimental.pallas import tpu_sc as plsc

DIM = 64
BATCH = 16384
REG_COEF = 1e-05
NC = 2
NS = 16
LANES = 16
NW = NC * NS
BPW = BATCH // NW
CHUNK = 128
NCHUNK = BPW // CHUNK
GPC = CHUNK // LANES


def _sc_body(idx_u, idx_p, idx_n, utab, itab, diff_hbm, sq_hbm,
             iu_v, ip_v, in_v,
             ru0, ru1, rp0, rp1, rn0, rn1,
             scores_v, sq_v, sem0, sem1):
    wid = lax.axis_index("s") * NC + lax.axis_index("c")

    pltpu.sync_copy(idx_u.at[wid], iu_v)
    pltpu.sync_copy(idx_p.at[wid], ip_v)
    pltpu.sync_copy(idx_n.at[wid], in_v)

    rbufs = [(ru0, rp0, rn0), (ru1, rp1, rn1)]
    sems = [sem0, sem1]

    def fire(c):
        ru, rp, rn = rbufs[c % 2]
        sem = sems[c % 2]
        return [
            pltpu.async_copy(utab.at[iu_v.at[c]], ru, sem),
            pltpu.async_copy(itab.at[ip_v.at[c]], rp, sem),
            pltpu.async_copy(itab.at[in_v.at[c]], rn, sem),
        ]

    sq_v[...] = jnp.zeros((LANES,), jnp.float32)
    iota = lax.iota(jnp.int32, LANES)

    pending = fire(0)
    for c in range(NCHUNK):
        nxt = fire(c + 1) if c + 1 < NCHUNK else []
        for cpd in pending:
            cpd.wait()
        pending = nxt
        ru, rp, rn = rbufs[c % 2]

        @pl.loop(0, GPC)
        def _group(g):
            row = g * LANES + iota
            pos = jnp.zeros((LANES,), jnp.float32)
            neg = jnp.zeros((LANES,), jnp.float32)
            sq = jnp.zeros((LANES,), jnp.float32)
            for d in range(DIM):
                col = jnp.full((LANES,), d, jnp.int32)
                u = plsc.load_gather(ru, [row, col])
                p = plsc.load_gather(rp, [row, col])
                n = plsc.load_gather(rn, [row, col])
                pos = pos + u * p
                neg = neg + u * n
                sq = sq + (u * u + p * p + n * n)
            scores_v[pl.ds(c * CHUNK + g * LANES, LANES)] = pos - neg
            sq_v[...] += sq

    pltpu.sync_copy(scores_v, diff_hbm.at[pl.ds(wid * BPW, BPW)])
    pltpu.sync_copy(sq_v, sq_hbm.at[wid])


def _loss_body(diff_ref, sq_ref, out_ref):
    d = diff_ref[...]
    sp = jnp.maximum(-d, 0.0) + jnp.log1p(jnp.exp(-jnp.abs(d)))
    out_ref[0] = jnp.sum(sp) * (1.0 / BATCH)
    out_ref[1] = (0.5 * REG_COEF) * jnp.sum(sq_ref[...])


@jax.jit
def kernel(userids, itemids_pos, itemids_neg, user_table, item_table):
    shp = (NW, NCHUNK, CHUNK)
    gidx = [x.astype(jnp.int32).reshape(shp)
            for x in (userids, itemids_pos, itemids_neg)]

    mesh = plsc.VectorSubcoreMesh(
        core_axis_name="c", subcore_axis_name="s",
        num_cores=NC, num_subcores=NS)

    cp = pltpu.CompilerParams()
    if "needs_layout_passes" in pltpu.CompilerParams.__dataclass_fields__:
        cp = dataclasses.replace(cp, needs_layout_passes=False)
    if "use_tc_tiling_on_sc" in pltpu.CompilerParams.__dataclass_fields__:
        cp = dataclasses.replace(cp, use_tc_tiling_on_sc=False)

    idx_t = pltpu.VMEM((NCHUNK, CHUNK), jnp.int32)
    row_t = pltpu.VMEM((CHUNK, DIM), jnp.float32)
    sc = pl.kernel(
        _sc_body,
        compiler_params=cp,
        out_type=[
            jax.ShapeDtypeStruct((BATCH,), jnp.float32),
            jax.ShapeDtypeStruct((NW, LANES), jnp.float32),
        ],
        mesh=mesh,
        scratch_types=[
            idx_t, idx_t, idx_t,
            row_t, row_t, row_t, row_t, row_t, row_t,
            pltpu.VMEM((BPW,), jnp.float32),
            pltpu.VMEM((LANES,), jnp.float32),
            pltpu.SemaphoreType.DMA,
            pltpu.SemaphoreType.DMA,
        ],
    )
    diff, sq = sc(*gidx, user_table, item_table)

    out = pl.pallas_call(
        _loss_body,
        out_shape=jax.ShapeDtypeStruct((2,), jnp.float32),
        out_specs=pl.BlockSpec(memory_space=pltpu.SMEM),
    )(diff.reshape(BATCH // 128, 128), sq)
    return out[0], out[1]
